# Initial kernel scaffold; baseline (speedup 1.0000x reference)
import jax
import jax.numpy as jnp
from jax.experimental import pallas as pl


def kernel(z, codebook):
    raise NotImplementedError("write your pallas kernel here")


# speedup vs baseline: 1.4474x; 1.4474x over previous
import functools

import numpy as np
import jax
import jax.numpy as jnp
from jax import lax
from jax.experimental import pallas as pl
from jax.experimental.pallas import tpu as pltpu
from jax.experimental.pallas import tpu_sc as plsc

_PATCH_NUMS = (1, 2, 4, 8, 16)
_BETA = 0.25
_VOCAB = 8192
_EMBED = 32
_BATCH = 256
_HW = 256

_BI = {1: 256, 2: 128, 4: 32, 8: 8, 16: 2}

_HIGHEST = jax.lax.Precision.HIGHEST


def _cubic_w(x, a=-0.75):
    ax = np.abs(x)
    ax2 = ax * ax
    ax3 = ax2 * ax
    w = np.where(ax <= 1.0, (a + 2.0) * ax3 - (a + 3.0) * ax2 + 1.0,
                 np.where(ax < 2.0, a * ax3 - 5.0 * a * ax2 + 8.0 * a * ax - 4.0 * a, 0.0))
    return w.astype(np.float32)


def _resize_mat(in_size, out_size):
    scale = in_size / out_size
    coords = (np.arange(out_size) + 0.5) * scale - 0.5
    i0 = np.floor(coords).astype(np.int64)
    M = np.zeros((out_size, in_size), dtype=np.float32)
    for off in range(-1, 3):
        idx = i0 + off
        w = _cubic_w(coords - idx)
        idxc = np.clip(idx, 0, in_size - 1)
        np.add.at(M, (np.arange(out_size), idxc), w)
    return M


def _down_mat_t(pn):
    s = 16 // pn
    Kd = np.zeros((pn * pn, _HW), dtype=np.float32)
    for py in range(pn):
        for px in range(pn):
            for h in range(py * s, (py + 1) * s):
                for w in range(px * s, (px + 1) * s):
                    Kd[py * pn + px, h * 16 + w] = 1.0 / (s * s)
    return np.ascontiguousarray(Kd.T)


def _up_mat_t(pn):
    M = _resize_mat(pn, 16)
    Ku = np.einsum('oh,pw->ophw', M, M).reshape(_HW, pn * pn).astype(np.float32)
    return np.ascontiguousarray(Ku.T)


def _argmin_body(pn, n_items, w_loss, fr_ref, cb_ref, cbt_ref, kdt_ref,
                 dprev_ref, idx_ref, dsum_ref):
    pn2 = pn * pn
    T = n_items * pn2
    x = fr_ref[...]
    if pn == 16:
        zf = jnp.swapaxes(x.reshape(n_items, _EMBED, _HW), 1, 2).reshape(T, _EMBED)
    elif pn == 1:
        x3 = x.reshape(n_items, _EMBED, _HW)
        zf = jnp.sum(x3, axis=2) * jnp.float32(1.0 / _HW)
    else:
        zs = lax.dot_general(x, kdt_ref[...], (((1,), (0,)), ((), ())),
                             preferred_element_type=jnp.float32,
                             precision=_HIGHEST)
        zf = jnp.swapaxes(zs.reshape(n_items, _EMBED, pn2), 1, 2).reshape(T, _EMBED)

    cb = cb_ref[...]
    cbt = cbt_ref[...]
    cn = jnp.sum(cb * cb, axis=1)[None, :]
    zn = jnp.sum(zf * zf, axis=1, keepdims=True)
    dims = (((1,), (0,)), ((), ()))
    if pn >= 8:
        zf_t = zf.T
        mmt = lax.dot_general(cb, zf_t, dims,
                              preferred_element_type=jnp.float32)
        znr = jnp.sum(zf_t * zf_t, axis=0, keepdims=True)
        cnc = jnp.sum(cb * cb, axis=1, keepdims=True)
        dt = (znr + cnc) - 2.0 * mmt
        idx_ref[0, 0, :] = jnp.argmin(dt, axis=0).astype(jnp.int32)
        part = jnp.sum(jnp.min(dt, axis=0)) * jnp.float32(w_loss)
    else:
        mm = lax.dot_general(zf, cbt, dims,
                             preferred_element_type=jnp.float32)
        d = (zn + cn) - 2.0 * mm
        idx_ref[0, 0, :] = jnp.argmin(d, axis=1).astype(jnp.int32)
        part = jnp.sum(jnp.min(d, axis=1)) * jnp.float32(w_loss)

    @pl.when(pl.program_id(0) == 0)
    def _init():
        dsum_ref[0, 0] = dprev_ref[0, 0] + part

    @pl.when(pl.program_id(0) > 0)
    def _acc():
        dsum_ref[0, 0] += part


def _argmin_call(fr, cb, cbt, kdt, dprev, pn, interpret=False):
    n_items = _BI[pn]
    nprog = _BATCH // n_items
    pn2 = pn * pn
    T = n_items * pn2
    body = functools.partial(_argmin_body, pn, n_items,
                             (1.0 + _BETA) / (5.0 * _BATCH * _EMBED * pn2))
    return pl.pallas_call(
        body,
        grid=(nprog,),
        in_specs=[
            pl.BlockSpec((n_items * _EMBED, _HW), lambda g: (g, 0)),
            pl.BlockSpec((_VOCAB, _EMBED), lambda g: (0, 0)),
            pl.BlockSpec((_EMBED, _VOCAB), lambda g: (0, 0)),
            pl.BlockSpec((_HW, pn2), lambda g: (0, 0)),
            pl.BlockSpec(memory_space=pltpu.SMEM),
        ],
        out_specs=[
            pl.BlockSpec((1, 1, T), lambda g: (g, 0, 0)),
            pl.BlockSpec(memory_space=pltpu.SMEM),
        ],
        out_shape=[
            jax.ShapeDtypeStruct((nprog, 1, T), jnp.int32),
            jax.ShapeDtypeStruct((1, 1), jnp.float32),
        ],
        interpret=interpret,
    )(fr, cb, cbt, kdt, dprev)


def _combine_body(pn, n_items, zq_ref, kut_ref, fr_ref, fh_ref,
                  fro_ref, fho_ref):
    pn2 = pn * pn
    zq = zq_ref[:, :_EMBED]
    if pn == 1:
        zq3 = jnp.broadcast_to(zq.reshape(n_items, 1, _EMBED),
                               (n_items, _HW, _EMBED))
        zq_cf = jnp.swapaxes(zq3, 1, 2).reshape(n_items * _EMBED, _HW)
        up = zq_cf * kut_ref[...]
    else:
        zq_cf = jnp.swapaxes(zq.reshape(n_items, pn2, _EMBED), 1, 2) \
                   .reshape(n_items * _EMBED, pn2)
        up = lax.dot_general(zq_cf, kut_ref[...], (((1,), (0,)), ((), ())),
                             preferred_element_type=jnp.float32,
                             precision=_HIGHEST)
    fho_ref[...] = fh_ref[...] + up
    fro_ref[...] = fr_ref[...] - up


def _combine_call(zq, kut, fr, fh, pn, interpret=False):
    n_items = _BI[pn]
    nprog = _BATCH // n_items
    pn2 = pn * pn
    return pl.pallas_call(
        functools.partial(_combine_body, pn, n_items),
        grid=(nprog,),
        in_specs=[
            pl.BlockSpec((n_items * pn2, 128), lambda g: (g, 0)),
            pl.BlockSpec((pn2, _HW), lambda g: (0, 0)),
            pl.BlockSpec((n_items * _EMBED, _HW), lambda g: (g, 0)),
            pl.BlockSpec((n_items * _EMBED, _HW), lambda g: (g, 0)),
        ],
        out_specs=[
            pl.BlockSpec((n_items * _EMBED, _HW), lambda g: (g, 0)),
            pl.BlockSpec((n_items * _EMBED, _HW), lambda g: (g, 0)),
        ],
        out_shape=[
            jax.ShapeDtypeStruct((_BATCH * _EMBED, _HW), jnp.float32),
            jax.ShapeDtypeStruct((_BATCH * _EMBED, _HW), jnp.float32),
        ],
        input_output_aliases={2: 0, 3: 1},
        interpret=interpret,
    )(zq, kut, fr, fh)


def _final_body(n_items, zq_ref, fh_ref, fho_ref):
    zq = zq_ref[:, :_EMBED]
    zq_cf = jnp.swapaxes(zq.reshape(n_items, _HW, _EMBED), 1, 2) \
               .reshape(n_items * _EMBED, _HW)
    fho_ref[...] = fh_ref[...] + zq_cf


def _final_call(zq, fh, interpret=False):
    n_items = _BI[16]
    nprog = _BATCH // n_items
    return pl.pallas_call(
        functools.partial(_final_body, n_items),
        grid=(nprog,),
        in_specs=[
            pl.BlockSpec((n_items * _HW, 128), lambda g: (g, 0)),
            pl.BlockSpec((n_items * _EMBED, _HW), lambda g: (g, 0)),
        ],
        out_specs=[
            pl.BlockSpec((n_items * _EMBED, _HW), lambda g: (g, 0)),
        ],
        out_shape=[
            jax.ShapeDtypeStruct((_BATCH * _EMBED, _HW), jnp.float32),
        ],
        input_output_aliases={1: 0},
        interpret=interpret,
    )(zq, fh)[0]


@functools.cache
def _make_gather(n_tokens):
    NW = 32
    bpw = n_tokens // NW
    csz = min(bpw, 128)
    nchunks = bpw // csz
    mesh = plsc.VectorSubcoreMesh(core_axis_name="c", subcore_axis_name="s")

    @functools.partial(
        pl.kernel, mesh=mesh,
        out_type=jax.ShapeDtypeStruct((n_tokens, 128), jnp.float32),
        scratch_types=[
            pltpu.VMEM((bpw,), jnp.int32),
            pltpu.VMEM((csz, 128), jnp.float32),
            pltpu.VMEM((csz, 128), jnp.float32),
            pltpu.SemaphoreType.DMA,
            pltpu.SemaphoreType.DMA,
        ],
    )
    def gather(table_hbm, idx_hbm, out_hbm, idx_v, rows_a, rows_b, sem_a, sem_b):
        wid = lax.axis_index("s") * 2 + lax.axis_index("c")
        base = wid * bpw
        pltpu.sync_copy(idx_hbm.at[pl.ds(base, bpw)], idx_v)
        bufs = (rows_a, rows_b)
        sems = (sem_a, sem_b)
        cps = [None, None]
        cps[0] = pltpu.async_copy(
            table_hbm.at[idx_v.at[pl.ds(0, csz)]], bufs[0], sems[0])
        for k in range(nchunks):
            nk = k + 1
            if nk < nchunks:
                cps[nk % 2] = pltpu.async_copy(
                    table_hbm.at[idx_v.at[pl.ds(nk * csz, csz)]],
                    bufs[nk % 2], sems[nk % 2])
            cps[k % 2].wait()
            pltpu.sync_copy(bufs[k % 2], out_hbm.at[pl.ds(base + k * csz, csz)])

    return gather


def kernel(z, codebook):
    b, c, h, w = z.shape
    fr = z.reshape(_BATCH * _EMBED, _HW)
    fh = jnp.zeros_like(fr)
    cb = codebook
    cbt = codebook.T
    cb_pad = jnp.pad(codebook, ((0, 0), (0, 128 - _EMBED)))
    dsum = jnp.zeros((1, 1), jnp.float32)
    ms_idx = []
    for i, pn in enumerate(_PATCH_NUMS):
        kdt = jnp.asarray(_down_mat_t(pn)) if pn not in (1, 16) \
            else jnp.zeros((_HW, pn * pn), jnp.float32)
        idx_blk, dsum = _argmin_call(fr, cb, cbt, kdt, dsum, pn)
        n_tok = _BATCH * pn * pn
        idx_flat = idx_blk.reshape(n_tok)
        zq = _make_gather(n_tok)(cb_pad, idx_flat)
        if pn != 16:
            kut = jnp.asarray(_up_mat_t(pn))
            fr, fh = _combine_call(zq, kut, fr, fh, pn)
        else:
            fh = _final_call(zq, fh)
        ms_idx.append(idx_flat.reshape(_BATCH, pn, pn))
    f_hat = fh.reshape(_BATCH, _EMBED, 16, 16)
    vq_loss = dsum[0, 0]
    return (f_hat, tuple(ms_idx), vq_loss)
